# baseline (speedup 1.0000x reference)
import functools

import jax
import jax.numpy as jnp
from jax import lax
from jax.experimental import pallas as pl
from jax.experimental.pallas import tpu as pltpu
from jax.experimental.pallas import tpu_sc as plsc

N = 10000
NP = 10240
E = 320000
EP = 327680
K = 80
KP = 128
NTILE = 32
RPT = NP // 16
R = 1024
G = NP // R


def _mesh():
    return plsc.VectorSubcoreMesh(core_axis_name="c", subcore_axis_name="s")


def _sc_degree(packed, zeros1, ones):
    epl = EP // NTILE
    nch = epl // K

    @functools.partial(
        pl.kernel,
        out_type=jax.ShapeDtypeStruct((2 * NP,), jnp.float32),
        mesh=_mesh(),
        scratch_types=[
            pltpu.VMEM((epl,), jnp.int32),
            pltpu.VMEM((K,), jnp.int32),
            pltpu.VMEM((K,), jnp.float32),
            pltpu.VMEM_SHARED((NP,), jnp.float32),
        ],
    )
    def k(packed_hbm, zeros_hbm, ones_hbm, deg_hbm, edges, dstbuf, onesbuf,
          slab):
        c = lax.axis_index("c")
        s = lax.axis_index("s")
        pltpu.sync_copy(packed_hbm.at[pl.ds((c * 16 + s) * epl, epl)], edges)
        pltpu.sync_copy(ones_hbm, onesbuf)
        pltpu.sync_copy(zeros_hbm, slab.at[pl.ds(s * RPT, RPT)])
        plsc.subcore_barrier()

        def body(q, carry):
            for i in range(K // 16):
                p = edges[pl.ds(q * K + i * 16, 16)]
                dstbuf[pl.ds(i * 16, 16)] = lax.shift_right_logical(p, 14)
            pltpu.sync_copy(onesbuf, slab.at[dstbuf], add=True)
            return carry

        lax.fori_loop(0, nch, body, 0)
        plsc.subcore_barrier()
        pltpu.sync_copy(slab.at[pl.ds(s * RPT, RPT)],
                        deg_hbm.at[pl.ds(c * NP + s * RPT, RPT)])

    return k(packed, zeros1, ones)


def _sc_prop(table, packed, zeros_rows, nb):
    epl = EP // (NTILE if nb == 1 else 16)
    ebuf = 10240
    halves = epl // ebuf
    nch = ebuf // KP
    NSET = 2
    ngrp, ntail = nch // NSET, nch % NSET
    nouts = 2 if nb == 1 else nb

    @functools.partial(
        pl.kernel,
        out_type=jax.ShapeDtypeStruct((nouts * NP, 128), jnp.float32),
        mesh=_mesh(),
        scratch_types=[
            pltpu.VMEM((ebuf,), jnp.int32),
            [pltpu.VMEM((KP,), jnp.int32) for _ in range(NSET)],
            [pltpu.VMEM((KP,), jnp.int32) for _ in range(NSET)],
            [pltpu.VMEM((KP, 128), jnp.float32) for _ in range(NSET)],
            pltpu.VMEM_SHARED((NP, 128), jnp.float32),
            [pltpu.SemaphoreType.DMA for _ in range(NSET)],
        ],
    )
    def k(table_hbm, packed_hbm, zeros_hbm, out_hbm,
          edges, idxs, dsts, rows, slab, sems):
        c = lax.axis_index("c")
        s = lax.axis_index("s")

        for j in range(1 if nb == 1 else nb // 2):
            if nb == 1:
                orow = c * NP
                off = None
            else:
                bid = c * (nb // 2) + j
                orow = bid * NP
                off = bid * NP

            def issue(q, t):
                for i in range(KP // 16):
                    sl = pl.ds(i * 16, 16)
                    p = edges[pl.ds(q * KP + i * 16, 16)]
                    v = p & 0x3FFF
                    idxs[t][sl] = v if off is None else v + off
                    dsts[t][sl] = lax.shift_right_logical(p, 14)
                pltpu.async_copy(table_hbm.at[idxs[t]], rows[t], sems[t])

            def drain(t):
                pltpu.make_async_copy(
                    table_hbm.at[idxs[t]], rows[t], sems[t]).wait()
                pltpu.sync_copy(rows[t], slab.at[dsts[t]], add=True)

            pltpu.sync_copy(zeros_hbm, slab.at[pl.ds(s * RPT, RPT)])
            plsc.subcore_barrier()

            for h in range(halves):
                if nb == 1:
                    ebase = (c * 16 + s) * epl + h * ebuf
                else:
                    ebase = s * epl + h * ebuf
                pltpu.sync_copy(packed_hbm.at[pl.ds(ebase, ebuf)], edges)
                for t in range(NSET):
                    issue(t, t)

                def body(g, carry):
                    for t in range(NSET):
                        q = NSET * g + t
                        drain(t)

                        @pl.when(q + NSET < nch)
                        def _():
                            issue(q + NSET, t)

                    return carry

                lax.fori_loop(0, ngrp, body, 0)
                for u in range(ntail):
                    drain((ngrp * NSET + u) % NSET)

            plsc.subcore_barrier()
            pltpu.sync_copy(slab.at[pl.ds(s * RPT, RPT)],
                            out_hbm.at[pl.ds(orow + s * RPT, RPT)])

    return k(table, packed, zeros_rows)


def _tc_prep(xp, dega, degb):
    def body(x_ref, da_ref, db_ref, dis_ref, y0_ref):
        deg = da_ref[...] + db_ref[...]
        dis = jnp.where(deg > 0, lax.rsqrt(jnp.maximum(deg, 1e-12)), 0.0)
        dis_ref[...] = dis
        y0_ref[...] = dis * x_ref[...]

    return pl.pallas_call(
        body,
        grid=(G,),
        in_specs=[pl.BlockSpec((R, 128), lambda i: (i, 0)),
                  pl.BlockSpec((R, 1), lambda i: (i, 0)),
                  pl.BlockSpec((R, 1), lambda i: (i, 0))],
        out_specs=[pl.BlockSpec((R, 1), lambda i: (i, 0)),
                   pl.BlockSpec((R, 128), lambda i: (i, 0))],
        out_shape=[jax.ShapeDtypeStruct((NP, 1), jnp.float32),
                   jax.ShapeDtypeStruct((NP, 128), jnp.float32)],
    )(xp, dega, degb)


def _tc_layer1(p1a, p1b, dis, w1, w2):
    def body(pa_ref, pb_ref, d_ref, w1_ref, w2_ref, out_ref):
        d = d_ref[...]
        p = (pa_ref[...] + pb_ref[...]) * d
        t = jnp.dot(p.astype(jnp.bfloat16), w1_ref[...],
                    preferred_element_type=jnp.float32)
        a = d * jnp.maximum(t, 0.0)
        y = jnp.dot(a.astype(jnp.bfloat16), w2_ref[...],
                    preferred_element_type=jnp.float32)
        for j in range(8):
            out_ref[j] = y[:, j * 128:(j + 1) * 128]

    return pl.pallas_call(
        body,
        grid=(G,),
        in_specs=[pl.BlockSpec((R, 128), lambda i: (i, 0)),
                  pl.BlockSpec((R, 128), lambda i: (i, 0)),
                  pl.BlockSpec((R, 1), lambda i: (i, 0)),
                  pl.BlockSpec((128, 1024), lambda i: (0, 0)),
                  pl.BlockSpec((1024, 1024), lambda i: (0, 0))],
        out_specs=pl.BlockSpec((8, R, 128), lambda i: (0, i, 0)),
        out_shape=jax.ShapeDtypeStruct((8, NP, 128), jnp.float32),
    )(p1a, p1b, dis, w1, w2)


def _tc_layer2(p2, dis, w3p):
    def body(p2_ref, d_ref, w3_ref, out_ref):
        d = d_ref[...]
        acc = jnp.zeros((R, 128), jnp.float32)
        for j in range(8):
            h = d * jnp.maximum(d * p2_ref[j], 0.0)
            acc = acc + jnp.dot(h.astype(jnp.bfloat16),
                                w3_ref[j * 128:(j + 1) * 128, :],
                                preferred_element_type=jnp.float32)
        out_ref[...] = acc

    return pl.pallas_call(
        body,
        grid=(G,),
        in_specs=[pl.BlockSpec((8, R, 128), lambda i: (0, i, 0)),
                  pl.BlockSpec((R, 1), lambda i: (i, 0)),
                  pl.BlockSpec((1024, 128), lambda i: (0, 0))],
        out_specs=pl.BlockSpec((R, 128), lambda i: (i, 0)),
        out_shape=jax.ShapeDtypeStruct((NP, 128), jnp.float32),
    )(p2, dis, w3p)


def _tc_final(p3a, p3b, dis):
    def body(pa_ref, pb_ref, d_ref, out_ref):
        out_ref[...] = d_ref[...] * (pa_ref[...] + pb_ref[...])

    return pl.pallas_call(
        body,
        grid=(G,),
        in_specs=[pl.BlockSpec((R, 128), lambda i: (i, 0)),
                  pl.BlockSpec((R, 128), lambda i: (i, 0)),
                  pl.BlockSpec((R, 1), lambda i: (i, 0))],
        out_specs=pl.BlockSpec((R, 128), lambda i: (i, 0)),
        out_shape=jax.ShapeDtypeStruct((NP, 128), jnp.float32),
    )(p3a, p3b, dis)


def kernel(X, edge_index, W1, W2, W3):
    src = edge_index[0]
    dst = edge_index[1]
    packed = jnp.bitwise_or(jnp.left_shift(dst, 14), src)
    packed = jnp.concatenate(
        [packed, jnp.full((EP - E,), ((NP - 1) << 14) | N, jnp.int32)])
    xp = jnp.pad(X, ((0, NP - N), (0, 0)))
    w3p = jnp.pad(W3, ((0, 0), (0, 128 - W3.shape[1])))
    zeros1 = jnp.zeros((RPT,), jnp.float32)
    zeros_rows = jnp.zeros((RPT, 128), jnp.float32)
    ones = jnp.ones((K,), jnp.float32)

    deg2 = _sc_degree(packed, zeros1, ones).reshape(2, NP, 1)
    dis, y0 = _tc_prep(xp, deg2[0], deg2[1])

    p1 = _sc_prop(y0, packed, zeros_rows, nb=1).reshape(2, NP, 128)
    y2 = _tc_layer1(p1[0], p1[1], dis,
                    W1.astype(jnp.bfloat16), W2.astype(jnp.bfloat16))

    p2 = _sc_prop(y2.reshape(8 * NP, 128), packed, zeros_rows, nb=8)
    y3 = _tc_layer2(p2.reshape(8, NP, 128), dis, w3p.astype(jnp.bfloat16))

    p3 = _sc_prop(y3, packed, zeros_rows, nb=1).reshape(2, NP, 128)
    out = _tc_final(p3[0], p3[1], dis)
    return out[:N, :W3.shape[1]]


# speedup vs baseline: 3.0691x; 3.0691x over previous
import functools

import jax
import jax.numpy as jnp
from jax import lax
from jax.experimental import pallas as pl
from jax.experimental.pallas import tpu as pltpu
from jax.experimental.pallas import tpu_sc as plsc

N = 10000
NP = 10240
E = 320000
K = 80
NTILE = 32
RPT = NP // 16
R = 1024
G = NP // R


def _mesh():
    return plsc.VectorSubcoreMesh(core_axis_name="c", subcore_axis_name="s")


def _sc_degree(packed, zeros1, ones):
    epl = E // NTILE
    nch = epl // K

    @functools.partial(
        pl.kernel,
        out_type=jax.ShapeDtypeStruct((2 * NP,), jnp.float32),
        mesh=_mesh(),
        scratch_types=[
            pltpu.VMEM((epl,), jnp.int32),
            pltpu.VMEM((K,), jnp.int32),
            pltpu.VMEM((K,), jnp.float32),
            pltpu.VMEM_SHARED((NP,), jnp.float32),
        ],
    )
    def k(packed_hbm, zeros_hbm, ones_hbm, deg_hbm, edges, dstbuf, onesbuf,
          slab):
        c = lax.axis_index("c")
        s = lax.axis_index("s")
        pltpu.sync_copy(packed_hbm.at[pl.ds((c * 16 + s) * epl, epl)], edges)
        pltpu.sync_copy(ones_hbm, onesbuf)
        pltpu.sync_copy(zeros_hbm, slab.at[pl.ds(s * RPT, RPT)])
        plsc.subcore_barrier()

        def body(q, carry):
            for i in range(K // 16):
                p = edges[pl.ds(q * K + i * 16, 16)]
                dstbuf[pl.ds(i * 16, 16)] = lax.shift_right_logical(p, 14)
            pltpu.sync_copy(onesbuf, slab.at[dstbuf], add=True)
            return carry

        lax.fori_loop(0, nch, body, 0)
        plsc.subcore_barrier()
        pltpu.sync_copy(slab.at[pl.ds(s * RPT, RPT)],
                        deg_hbm.at[pl.ds(c * NP + s * RPT, RPT)])

    return k(packed, zeros1, ones)


def _sc_prop(table, packed, zeros_rows, nb):
    epl = E // (NTILE if nb == 1 else 16)
    ebuf = 10000
    halves = epl // ebuf
    nch = ebuf // K
    NSET = 3
    ngrp, ntail = nch // NSET, nch % NSET
    nouts = 2 if nb == 1 else nb

    @functools.partial(
        pl.kernel,
        out_type=jax.ShapeDtypeStruct((nouts * NP, 128), jnp.float32),
        mesh=_mesh(),
        scratch_types=[
            pltpu.VMEM((ebuf,), jnp.int32),
            [pltpu.VMEM((K,), jnp.int32) for _ in range(NSET)],
            [pltpu.VMEM((K,), jnp.int32) for _ in range(NSET)],
            [pltpu.VMEM((K, 128), jnp.float32) for _ in range(NSET)],
            pltpu.VMEM_SHARED((NP, 128), jnp.float32),
            [pltpu.SemaphoreType.DMA for _ in range(NSET)],
        ],
    )
    def k(table_hbm, packed_hbm, zeros_hbm, out_hbm,
          edges, idxs, dsts, rows, slab, sems):
        c = lax.axis_index("c")
        s = lax.axis_index("s")

        for j in range(1 if nb == 1 else nb // 2):
            if nb == 1:
                orow = c * NP
                off = None
            else:
                bid = c * (nb // 2) + j
                orow = bid * NP
                off = bid * NP

            def issue(q, t):
                for i in range(K // 16):
                    sl = pl.ds(i * 16, 16)
                    p = edges[pl.ds(q * K + i * 16, 16)]
                    v = p & 0x3FFF
                    idxs[t][sl] = v if off is None else v + off
                    dsts[t][sl] = lax.shift_right_logical(p, 14)
                pltpu.async_copy(table_hbm.at[idxs[t]], rows[t], sems[t])

            def drain(t):
                pltpu.make_async_copy(
                    table_hbm.at[idxs[t]], rows[t], sems[t]).wait()
                pltpu.sync_copy(rows[t], slab.at[dsts[t]], add=True)

            pltpu.sync_copy(zeros_hbm, slab.at[pl.ds(s * RPT, RPT)])
            plsc.subcore_barrier()

            for h in range(halves):
                if nb == 1:
                    ebase = (c * 16 + s) * epl + h * ebuf
                else:
                    ebase = s * epl + h * ebuf
                pltpu.sync_copy(packed_hbm.at[pl.ds(ebase, ebuf)], edges)
                for t in range(NSET):
                    issue(t, t)

                def body(g, carry):
                    for t in range(NSET):
                        q = NSET * g + t
                        drain(t)

                        @pl.when(q + NSET < nch)
                        def _():
                            issue(q + NSET, t)

                    return carry

                lax.fori_loop(0, ngrp, body, 0)
                for u in range(ntail):
                    drain((ngrp * NSET + u) % NSET)

            plsc.subcore_barrier()
            pltpu.sync_copy(slab.at[pl.ds(s * RPT, RPT)],
                            out_hbm.at[pl.ds(orow + s * RPT, RPT)])

    return k(table, packed, zeros_rows)


def _tc_prep(xp, dega, degb):
    def body(x_ref, da_ref, db_ref, dis_ref, y0_ref):
        deg = da_ref[...] + db_ref[...]
        dis = jnp.where(deg > 0, lax.rsqrt(jnp.maximum(deg, 1e-12)), 0.0)
        dis_ref[...] = dis
        y0_ref[...] = dis * x_ref[...]

    return pl.pallas_call(
        body,
        grid=(G,),
        in_specs=[pl.BlockSpec((R, 128), lambda i: (i, 0)),
                  pl.BlockSpec((R, 1), lambda i: (i, 0)),
                  pl.BlockSpec((R, 1), lambda i: (i, 0))],
        out_specs=[pl.BlockSpec((R, 1), lambda i: (i, 0)),
                   pl.BlockSpec((R, 128), lambda i: (i, 0))],
        out_shape=[jax.ShapeDtypeStruct((NP, 1), jnp.float32),
                   jax.ShapeDtypeStruct((NP, 128), jnp.float32)],
    )(xp, dega, degb)


def _tc_layer1(p1a, p1b, dis, w1, w2):
    def body(pa_ref, pb_ref, d_ref, w1_ref, w2_ref, out_ref):
        d = d_ref[...]
        p = (pa_ref[...] + pb_ref[...]) * d
        t = jnp.dot(p.astype(jnp.bfloat16), w1_ref[...],
                    preferred_element_type=jnp.float32)
        a = d * jnp.maximum(t, 0.0)
        y = jnp.dot(a.astype(jnp.bfloat16), w2_ref[...],
                    preferred_element_type=jnp.float32)
        for j in range(8):
            out_ref[j] = y[:, j * 128:(j + 1) * 128]

    return pl.pallas_call(
        body,
        grid=(G,),
        in_specs=[pl.BlockSpec((R, 128), lambda i: (i, 0)),
                  pl.BlockSpec((R, 128), lambda i: (i, 0)),
                  pl.BlockSpec((R, 1), lambda i: (i, 0)),
                  pl.BlockSpec((128, 1024), lambda i: (0, 0)),
                  pl.BlockSpec((1024, 1024), lambda i: (0, 0))],
        out_specs=pl.BlockSpec((8, R, 128), lambda i: (0, i, 0)),
        out_shape=jax.ShapeDtypeStruct((8, NP, 128), jnp.float32),
    )(p1a, p1b, dis, w1, w2)


def _tc_layer2(p2, dis, w3p):
    def body(p2_ref, d_ref, w3_ref, out_ref):
        d = d_ref[...]
        acc = jnp.zeros((R, 128), jnp.float32)
        for j in range(8):
            h = d * jnp.maximum(d * p2_ref[j], 0.0)
            acc = acc + jnp.dot(h.astype(jnp.bfloat16),
                                w3_ref[j * 128:(j + 1) * 128, :],
                                preferred_element_type=jnp.float32)
        out_ref[...] = acc

    return pl.pallas_call(
        body,
        grid=(G,),
        in_specs=[pl.BlockSpec((8, R, 128), lambda i: (0, i, 0)),
                  pl.BlockSpec((R, 1), lambda i: (i, 0)),
                  pl.BlockSpec((1024, 128), lambda i: (0, 0))],
        out_specs=pl.BlockSpec((R, 128), lambda i: (i, 0)),
        out_shape=jax.ShapeDtypeStruct((NP, 128), jnp.float32),
    )(p2, dis, w3p)


def _tc_final(p3a, p3b, dis):
    def body(pa_ref, pb_ref, d_ref, out_ref):
        out_ref[...] = d_ref[...] * (pa_ref[...] + pb_ref[...])

    return pl.pallas_call(
        body,
        grid=(G,),
        in_specs=[pl.BlockSpec((R, 128), lambda i: (i, 0)),
                  pl.BlockSpec((R, 128), lambda i: (i, 0)),
                  pl.BlockSpec((R, 1), lambda i: (i, 0))],
        out_specs=pl.BlockSpec((R, 128), lambda i: (i, 0)),
        out_shape=jax.ShapeDtypeStruct((NP, 128), jnp.float32),
    )(p3a, p3b, dis)


def kernel(X, edge_index, W1, W2, W3):
    src = edge_index[0]
    dst = edge_index[1]
    packed = jnp.bitwise_or(jnp.left_shift(dst, 14), src)
    xp = jnp.pad(X, ((0, NP - N), (0, 0)))
    w3p = jnp.pad(W3, ((0, 0), (0, 128 - W3.shape[1])))
    zeros1 = jnp.zeros((RPT,), jnp.float32)
    zeros_rows = jnp.zeros((RPT, 128), jnp.float32)
    ones = jnp.ones((K,), jnp.float32)

    deg2 = _sc_degree(packed, zeros1, ones).reshape(2, NP, 1)
    dis, y0 = _tc_prep(xp, deg2[0], deg2[1])

    p1 = _sc_prop(y0, packed, zeros_rows, nb=1).reshape(2, NP, 128)
    y2 = _tc_layer1(p1[0], p1[1], dis,
                    W1.astype(jnp.bfloat16), W2.astype(jnp.bfloat16))

    p2 = _sc_prop(y2.reshape(8 * NP, 128), packed, zeros_rows, nb=8)
    y3 = _tc_layer2(p2.reshape(8, NP, 128), dis, w3p.astype(jnp.bfloat16))

    p3 = _sc_prop(y3, packed, zeros_rows, nb=1).reshape(2, NP, 128)
    out = _tc_final(p3[0], p3[1], dis)
    return out[:N, :W3.shape[1]]
